# baseline (speedup 1.0000x reference)
import functools

import jax
import jax.numpy as jnp
from jax import lax
from jax.experimental import pallas as pl
from jax.experimental.pallas import tpu as pltpu
from jax.experimental.pallas import tpu_sc as plsc

B, S, D = 1, 2048, 1024
E, K, H = 8, 2, 1024
N = B * S
TT = 256
NPAD = N * K + E * TT
NT2 = NPAD // TT
NEG = -1e30

NW = 32
GCH = 64
CCH = 32


def _router_body(wg_ref, x_ref, logits_ref, sel_ref, rw_ref):
    xt = x_ref[...]
    lt = lax.dot_general(
        wg_ref[...], xt, (((1,), (1,)), ((), ())),
        preferred_element_type=jnp.float32)
    idx = lax.broadcasted_iota(jnp.int32, (E, TT), 0)
    m1 = jnp.max(lt, axis=0, keepdims=True)
    a1 = jnp.min(jnp.where(lt == m1, idx, E), axis=0, keepdims=True)
    lt2 = jnp.where(idx == a1, NEG, lt)
    m2 = jnp.max(lt2, axis=0, keepdims=True)
    a2 = jnp.min(jnp.where(lt2 == m2, idx, E), axis=0, keepdims=True)
    e2 = jnp.exp(m2 - m1)
    denom = 1.0 + e2
    logits_ref[...] = lt
    sel_ref[...] = jnp.concatenate([a1, a2], axis=0)
    rw_ref[...] = jnp.concatenate([1.0 / denom, e2 / denom], axis=0)


def _router(xf, Wg):
    nt = N // TT
    return pl.pallas_call(
        _router_body,
        grid=(nt,),
        in_specs=[
            pl.BlockSpec((E, D), lambda t: (0, 0)),
            pl.BlockSpec((TT, D), lambda t: (t, 0)),
        ],
        out_specs=[
            pl.BlockSpec((E, TT), lambda t: (0, t)),
            pl.BlockSpec((K, TT), lambda t: (0, t)),
            pl.BlockSpec((K, TT), lambda t: (0, t)),
        ],
        out_shape=[
            jax.ShapeDtypeStruct((E, N), jnp.float32),
            jax.ShapeDtypeStruct((K, N), jnp.int32),
            jax.ShapeDtypeStruct((K, N), jnp.float32),
        ],
    )(Wg, xf)


def _dispatch_meta(selT, rwT):
    sel_flat = selT.reshape(-1)
    w_flat = rwT.reshape(-1)
    tok_flat = jnp.tile(jnp.arange(N, dtype=jnp.int32), K)
    onehot = (sel_flat[:, None] == jnp.arange(E, dtype=jnp.int32)[None, :])
    oh = onehot.astype(jnp.int32)
    counts = jnp.sum(oh, axis=0)
    rank = jnp.sum((jnp.cumsum(oh, axis=0) - 1) * oh, axis=1)
    pcounts = ((counts + TT - 1) // TT) * TT
    pcum = jnp.cumsum(pcounts)
    pstart = pcum - pcounts
    dest = pstart[sel_flat] + rank
    gidx = jnp.zeros((NPAD,), jnp.int32).at[dest].set(tok_flat)
    wslot = jnp.zeros((NPAD,), jnp.float32).at[dest].set(w_flat)
    invr = dest.astype(jnp.int32).reshape(K, N)
    tile_starts = jnp.arange(NT2, dtype=jnp.int32) * TT
    te = jnp.searchsorted(pcum, tile_starts, side='right')
    active = (tile_starts < pcum[-1]).astype(jnp.int32)
    te = jnp.minimum(te, E - 1).astype(jnp.int32)
    return gidx, wslot.reshape(NPAD, 1), invr, te, active


def _sc_gather(xb32, gidx):
    D2 = D // 2
    rpw = NPAD // NW
    nch = rpw // GCH
    mesh = plsc.VectorSubcoreMesh(core_axis_name="c", subcore_axis_name="s")

    @functools.partial(
        pl.kernel,
        out_type=jax.ShapeDtypeStruct((NPAD, D2), jnp.int32),
        mesh=mesh,
        scratch_types=[
            pltpu.VMEM((GCH,), jnp.int32),
            pltpu.VMEM((GCH,), jnp.int32),
            pltpu.VMEM((GCH, D2), jnp.int32),
            pltpu.VMEM((GCH, D2), jnp.int32),
            pltpu.SemaphoreType.DMA,
            pltpu.SemaphoreType.DMA,
            pltpu.SemaphoreType.DMA,
            pltpu.SemaphoreType.DMA,
        ],
    )
    def k(x_hbm, idx_hbm, xs_hbm, i0, i1, r0, r1, sg0, sg1, sw0, sw1):
        wid = lax.axis_index("s") * 2 + lax.axis_index("c")
        base = wid * rpw
        bufs = ((i0, r0, sg0, sw0), (i1, r1, sg1, sw1))
        g = [None] * nch
        wb = [None] * nch
        for c in range(nch):
            i_v, r_v, sg, _ = bufs[c & 1]
            if c >= 2:
                wb[c - 2].wait()
            pltpu.sync_copy(idx_hbm.at[pl.ds(base + c * GCH, GCH)], i_v)
            g[c] = pltpu.async_copy(x_hbm.at[i_v], r_v, sg)
            if c >= 1:
                pc = c - 1
                _, pr, _, psw = bufs[pc & 1]
                g[pc].wait()
                wb[pc] = pltpu.async_copy(
                    pr, xs_hbm.at[pl.ds(base + pc * GCH, GCH)], psw)
        lc = nch - 1
        _, lr, _, lsw = bufs[lc & 1]
        g[lc].wait()
        wb[lc] = pltpu.async_copy(
            lr, xs_hbm.at[pl.ds(base + lc * GCH, GCH)], lsw)
        if nch >= 2:
            wb[nch - 2].wait()
        wb[lc].wait()

    return k(xb32, gidx)


def _expert_body(te_ref, act_ref, xs_ref, w1_ref, w2_ref, ws_ref, ys_ref):
    t = pl.program_id(0)

    @pl.when(act_ref[t] == 1)
    def _():
        xt = xs_ref[...]
        g = lax.dot_general(
            xt, w1_ref[0, 0], (((1,), (1,)), ((), ())),
            preferred_element_type=jnp.float32)
        l = lax.dot_general(
            xt, w1_ref[0, 1], (((1,), (1,)), ((), ())),
            preferred_element_type=jnp.float32)
        a = (g * lax.logistic(g) * l).astype(jnp.bfloat16)
        oe = lax.dot_general(
            a, w2_ref[0], (((1,), (1,)), ((), ())),
            preferred_element_type=jnp.float32)
        ys_ref[...] = ws_ref[...] * oe

    @pl.when(act_ref[t] == 0)
    def _():
        ys_ref[...] = jnp.zeros_like(ys_ref)


def _experts(xs, W1r, W2, wslot, te, active):
    grid_spec = pltpu.PrefetchScalarGridSpec(
        num_scalar_prefetch=2,
        grid=(NT2,),
        in_specs=[
            pl.BlockSpec((TT, D), lambda t, te_r, ac_r: (t, 0)),
            pl.BlockSpec((1, 2, H, D), lambda t, te_r, ac_r: (te_r[t], 0, 0, 0)),
            pl.BlockSpec((1, D, H), lambda t, te_r, ac_r: (te_r[t], 0, 0)),
            pl.BlockSpec((TT, 1), lambda t, te_r, ac_r: (t, 0)),
        ],
        out_specs=pl.BlockSpec((TT, D), lambda t, te_r, ac_r: (t, 0)),
    )
    return pl.pallas_call(
        _expert_body,
        grid_spec=grid_spec,
        out_shape=jax.ShapeDtypeStruct((NPAD, D), jnp.float32),
    )(te, active, xs, W1r, W2, wslot)


def _sc_combine(ys, i0, i1):
    tpw = N // NW
    mesh = plsc.VectorSubcoreMesh(core_axis_name="c", subcore_axis_name="s")

    @functools.partial(
        pl.kernel,
        out_type=jax.ShapeDtypeStruct((N, D), jnp.float32),
        mesh=mesh,
        scratch_types=[
            pltpu.VMEM((CCH,), jnp.int32),
            pltpu.VMEM((CCH,), jnp.int32),
            pltpu.VMEM((CCH, D), jnp.float32),
            pltpu.VMEM((CCH, D), jnp.float32),
            pltpu.SemaphoreType.DMA,
            pltpu.SemaphoreType.DMA,
        ],
    )
    def k(ys_hbm, i0_hbm, i1_hbm, out_hbm, i0_v, i1_v, g0, g1, sem0, sem1):
        wid = lax.axis_index("s") * 2 + lax.axis_index("c")
        base = wid * tpw
        for c in range(tpw // CCH):
            off = base + c * CCH
            pltpu.sync_copy(i0_hbm.at[pl.ds(off, CCH)], i0_v)
            pltpu.sync_copy(i1_hbm.at[pl.ds(off, CCH)], i1_v)
            cp0 = pltpu.async_copy(ys_hbm.at[i0_v], g0, sem0)
            cp1 = pltpu.async_copy(ys_hbm.at[i1_v], g1, sem1)
            cp0.wait()
            cp1.wait()

            def add_row(r, carry):
                for j in range(D // 16):
                    sl = pl.ds(j * 16, 16)
                    g0[r, sl] = g0[r, sl] + g1[r, sl]
                return carry

            lax.fori_loop(0, CCH, add_row, 0)
            pltpu.sync_copy(g0, out_hbm.at[pl.ds(off, CCH)])

    return k(ys, i0, i1)


@jax.jit
def kernel(x, Wg, W1, W2):
    xf = x.reshape(N, D)
    logitsT, selT, rwT = _router(xf, Wg)
    gidx, wslot, invr, te, active = _dispatch_meta(selT, rwT)
    xb = xf.astype(jnp.bfloat16)
    xb32 = lax.bitcast_convert_type(xb.reshape(N, D // 2, 2), jnp.int32)
    xs32 = _sc_gather(xb32, gidx)
    xs = lax.bitcast_convert_type(xs32, jnp.bfloat16).reshape(NPAD, D)
    W1b = W1.reshape(E, 2, H, D).astype(jnp.bfloat16)
    W2b = W2.astype(jnp.bfloat16)
    ys = _experts(xs, W1b, W2b, wslot, te, active)
    out = _sc_combine(ys, invr[0], invr[1])
    return out.reshape(B, S, D), logitsT.T.reshape(B, S, E)


# speedup vs baseline: 1.3248x; 1.3248x over previous
import functools

import jax
import jax.numpy as jnp
from jax import lax
from jax.experimental import pallas as pl
from jax.experimental.pallas import tpu as pltpu
from jax.experimental.pallas import tpu_sc as plsc

B, S, D = 1, 2048, 1024
E, K, H = 8, 2, 1024
N = B * S
TT = 256
NPAD = N * K + E * TT
NT2 = NPAD // TT
NEG = -1e30

NW = 32
SCH = 64
CCH = 32


def _router_body(wg_ref, x_ref, logits_ref, sel_ref, rw_ref):
    xt = x_ref[...]
    lt = lax.dot_general(
        wg_ref[...], xt, (((1,), (1,)), ((), ())),
        preferred_element_type=jnp.float32)
    idx = lax.broadcasted_iota(jnp.int32, (E, TT), 0)
    m1 = jnp.max(lt, axis=0, keepdims=True)
    a1 = jnp.min(jnp.where(lt == m1, idx, E), axis=0, keepdims=True)
    lt2 = jnp.where(idx == a1, NEG, lt)
    m2 = jnp.max(lt2, axis=0, keepdims=True)
    a2 = jnp.min(jnp.where(lt2 == m2, idx, E), axis=0, keepdims=True)
    e2 = jnp.exp(m2 - m1)
    denom = 1.0 + e2
    logits_ref[...] = lt
    sel_ref[...] = jnp.concatenate([a1, a2], axis=0)
    rw_ref[...] = jnp.concatenate([1.0 / denom, e2 / denom], axis=0)


def _router(xf, Wg):
    nt = N // TT
    return pl.pallas_call(
        _router_body,
        grid=(nt,),
        in_specs=[
            pl.BlockSpec((E, D), lambda t: (0, 0)),
            pl.BlockSpec((TT, D), lambda t: (t, 0)),
        ],
        out_specs=[
            pl.BlockSpec((E, TT), lambda t: (0, t)),
            pl.BlockSpec((K, TT), lambda t: (0, t)),
            pl.BlockSpec((K, TT), lambda t: (0, t)),
        ],
        out_shape=[
            jax.ShapeDtypeStruct((E, N), jnp.float32),
            jax.ShapeDtypeStruct((K, N), jnp.int32),
            jax.ShapeDtypeStruct((K, N), jnp.float32),
        ],
    )(Wg, xf)


def _dispatch_meta(selT, rwT):
    sel_flat = selT.reshape(-1)
    w_flat = rwT.reshape(-1)
    onehot = (sel_flat[:, None] == jnp.arange(E, dtype=jnp.int32)[None, :])
    oh = onehot.astype(jnp.int32)
    counts = jnp.sum(oh, axis=0)
    rank = jnp.sum((jnp.cumsum(oh, axis=0) - 1) * oh, axis=1)
    pcounts = ((counts + TT - 1) // TT) * TT
    pcum = jnp.cumsum(pcounts)
    pstart = pcum - pcounts
    pstart_sel = jnp.sum(oh * pstart[None, :], axis=1)
    dest = (pstart_sel + rank).astype(jnp.int32)
    invr = dest.reshape(K, N)
    wwide = jnp.broadcast_to(w_flat[:, None], (K * N, 128))
    tile_starts = jnp.arange(NT2, dtype=jnp.int32) * TT
    te = jnp.sum((pcum[None, :] <= tile_starts[:, None]).astype(jnp.int32),
                 axis=1)
    active = (tile_starts < pcum[E - 1]).astype(jnp.int32)
    te = jnp.minimum(te, E - 1).astype(jnp.int32)
    return dest, wwide, invr, te, active


def _sc_dispatch(xb32, dest3, wwide):
    D2 = D // 2
    spw = (K * N) // NW
    nch = spw // SCH
    mesh = plsc.VectorSubcoreMesh(core_axis_name="c", subcore_axis_name="s")

    @functools.partial(
        pl.kernel,
        out_type=[
            jax.ShapeDtypeStruct((NPAD, D2), jnp.int32),
            jax.ShapeDtypeStruct((NPAD, 128), jnp.float32),
        ],
        mesh=mesh,
        scratch_types=[
            pltpu.VMEM((SCH,), jnp.int32),
            pltpu.VMEM((SCH, D2), jnp.int32),
            pltpu.VMEM((SCH, 128), jnp.float32),
            pltpu.SemaphoreType.DMA,
            pltpu.SemaphoreType.DMA,
        ],
    )
    def k(x_hbm, d_hbm, w_hbm, xs_hbm, ws_hbm, idx_v, rows_v, w_v, s0, s1):
        wid = lax.axis_index("s") * 2 + lax.axis_index("c")
        base = wid * spw
        tok = base - (base // N) * N
        for c in range(nch):
            off = c * SCH
            pltpu.sync_copy(d_hbm.at[wid, c], idx_v)
            pltpu.sync_copy(x_hbm.at[pl.ds(tok + off, SCH)], rows_v)
            pltpu.sync_copy(w_hbm.at[pl.ds(base + off, SCH)], w_v)
            cpx = pltpu.async_copy(rows_v, xs_hbm.at[idx_v], s0)
            cpw = pltpu.async_copy(w_v, ws_hbm.at[idx_v], s1)
            cpx.wait()
            cpw.wait()

    return k(xb32, dest3, wwide)


def _expert_body(te_ref, act_ref, xs_ref, w1_ref, w2_ref, ws_ref, ys_ref):
    t = pl.program_id(0)

    @pl.when(act_ref[t] == 1)
    def _():
        xt = xs_ref[...]
        g = lax.dot_general(
            xt, w1_ref[0, 0], (((1,), (1,)), ((), ())),
            preferred_element_type=jnp.float32)
        l = lax.dot_general(
            xt, w1_ref[0, 1], (((1,), (1,)), ((), ())),
            preferred_element_type=jnp.float32)
        a = (g * lax.logistic(g) * l).astype(jnp.bfloat16)
        oe = lax.dot_general(
            a, w2_ref[0], (((1,), (1,)), ((), ())),
            preferred_element_type=jnp.float32)
        ys_ref[...] = ws_ref[...] * oe

    @pl.when(act_ref[t] == 0)
    def _():
        ys_ref[...] = jnp.zeros_like(ys_ref)


def _experts(xs, W1r, W2, wslot, te, active):
    grid_spec = pltpu.PrefetchScalarGridSpec(
        num_scalar_prefetch=2,
        grid=(NT2,),
        in_specs=[
            pl.BlockSpec((TT, D), lambda t, te_r, ac_r: (t, 0)),
            pl.BlockSpec((1, 2, H, D), lambda t, te_r, ac_r: (te_r[t], 0, 0, 0)),
            pl.BlockSpec((1, D, H), lambda t, te_r, ac_r: (te_r[t], 0, 0)),
            pl.BlockSpec((TT, 1), lambda t, te_r, ac_r: (t, 0)),
        ],
        out_specs=pl.BlockSpec((TT, D), lambda t, te_r, ac_r: (t, 0)),
    )
    return pl.pallas_call(
        _expert_body,
        grid_spec=grid_spec,
        out_shape=jax.ShapeDtypeStruct((NPAD, D), jnp.float32),
    )(te, active, xs, W1r, W2, wslot)


def _sc_combine(ys, i0, i1):
    tpw = N // NW
    mesh = plsc.VectorSubcoreMesh(core_axis_name="c", subcore_axis_name="s")

    @functools.partial(
        pl.kernel,
        out_type=jax.ShapeDtypeStruct((N, D), jnp.float32),
        mesh=mesh,
        scratch_types=[
            pltpu.VMEM((CCH,), jnp.int32),
            pltpu.VMEM((CCH,), jnp.int32),
            pltpu.VMEM((CCH, D), jnp.float32),
            pltpu.VMEM((CCH, D), jnp.float32),
            pltpu.SemaphoreType.DMA,
            pltpu.SemaphoreType.DMA,
        ],
    )
    def k(ys_hbm, i0_hbm, i1_hbm, out_hbm, i0_v, i1_v, g0, g1, sem0, sem1):
        wid = lax.axis_index("s") * 2 + lax.axis_index("c")
        base = wid * tpw
        for c in range(tpw // CCH):
            off = base + c * CCH
            pltpu.sync_copy(i0_hbm.at[pl.ds(off, CCH)], i0_v)
            pltpu.sync_copy(i1_hbm.at[pl.ds(off, CCH)], i1_v)
            cp0 = pltpu.async_copy(ys_hbm.at[i0_v], g0, sem0)
            cp1 = pltpu.async_copy(ys_hbm.at[i1_v], g1, sem1)
            cp0.wait()
            cp1.wait()

            def add_row(r, carry):
                for j in range(D // 16):
                    sl = pl.ds(j * 16, 16)
                    g0[r, sl] = g0[r, sl] + g1[r, sl]
                return carry

            lax.fori_loop(0, CCH, add_row, 0)
            pltpu.sync_copy(g0, out_hbm.at[pl.ds(off, CCH)])

    return k(ys, i0, i1)


@jax.jit
def kernel(x, Wg, W1, W2):
    xf = x.reshape(N, D)
    logitsT, selT, rwT = _router(xf, Wg)
    dest, wwide, invr, te, active = _dispatch_meta(selT, rwT)
    xb = xf.astype(jnp.bfloat16)
    xb32 = lax.bitcast_convert_type(xb.reshape(N, D // 2, 2), jnp.int32)
    dest3 = dest.reshape(NW, (K * N) // (NW * SCH), SCH)
    xs32, ws = _sc_dispatch(xb32, dest3, wwide)
    xs = lax.bitcast_convert_type(xs32, jnp.bfloat16).reshape(NPAD, D)
    wslot = lax.slice(ws, (0, 0), (NPAD, 1))
    W1b = W1.reshape(E, 2, H, D).astype(jnp.bfloat16)
    W2b = W2.astype(jnp.bfloat16)
    ys = _experts(xs, W1b, W2b, wslot, te, active)
    out = _sc_combine(ys, invr[0], invr[1])
    return out.reshape(B, S, D), logitsT.T.reshape(B, S, E)


# speedup vs baseline: 1.3394x; 1.0110x over previous
import functools

import jax
import jax.numpy as jnp
from jax import lax
from jax.experimental import pallas as pl
from jax.experimental.pallas import tpu as pltpu
from jax.experimental.pallas import tpu_sc as plsc

B, S, D = 1, 2048, 1024
E, K, H = 8, 2, 1024
N = B * S
TT = 256
NPAD = N * K + E * TT
NT2 = NPAD // TT
NEG = -1e30

NW = 32
SCH = 64
CCH = 16


def _router_body(wg_ref, x_ref, logits_ref, sel_ref, rw_ref):
    xt = x_ref[...]
    lt = lax.dot_general(
        wg_ref[...], xt, (((1,), (1,)), ((), ())),
        preferred_element_type=jnp.float32)
    idx = lax.broadcasted_iota(jnp.int32, (E, TT), 0)
    m1 = jnp.max(lt, axis=0, keepdims=True)
    a1 = jnp.min(jnp.where(lt == m1, idx, E), axis=0, keepdims=True)
    lt2 = jnp.where(idx == a1, NEG, lt)
    m2 = jnp.max(lt2, axis=0, keepdims=True)
    a2 = jnp.min(jnp.where(lt2 == m2, idx, E), axis=0, keepdims=True)
    e2 = jnp.exp(m2 - m1)
    denom = 1.0 + e2
    logits_ref[...] = lt
    sel_ref[...] = jnp.concatenate([a1, a2], axis=0)
    rw_ref[...] = jnp.concatenate([1.0 / denom, e2 / denom], axis=0)


def _router(xf, Wg):
    nt = N // TT
    return pl.pallas_call(
        _router_body,
        grid=(nt,),
        in_specs=[
            pl.BlockSpec((E, D), lambda t: (0, 0)),
            pl.BlockSpec((TT, D), lambda t: (t, 0)),
        ],
        out_specs=[
            pl.BlockSpec((E, TT), lambda t: (0, t)),
            pl.BlockSpec((K, TT), lambda t: (0, t)),
            pl.BlockSpec((K, TT), lambda t: (0, t)),
        ],
        out_shape=[
            jax.ShapeDtypeStruct((E, N), jnp.float32),
            jax.ShapeDtypeStruct((K, N), jnp.int32),
            jax.ShapeDtypeStruct((K, N), jnp.float32),
        ],
    )(Wg, xf)


def _dispatch_meta(selT, rwT):
    sel_flat = selT.reshape(-1)
    w_flat = rwT.reshape(-1)
    onehot = (sel_flat[:, None] == jnp.arange(E, dtype=jnp.int32)[None, :])
    oh = onehot.astype(jnp.int32)
    counts = jnp.sum(oh, axis=0)
    rank = jnp.sum((jnp.cumsum(oh, axis=0) - 1) * oh, axis=1)
    pcounts = ((counts + TT - 1) // TT) * TT
    pcum = jnp.cumsum(pcounts)
    pstart = pcum - pcounts
    pstart_sel = jnp.sum(oh * pstart[None, :], axis=1)
    dest = (pstart_sel + rank).astype(jnp.int32)
    invr = dest.reshape(K, N)
    wwide = jnp.broadcast_to(w_flat[:, None], (K * N, 128))
    tile_starts = jnp.arange(NT2, dtype=jnp.int32) * TT
    te = jnp.sum((pcum[None, :] <= tile_starts[:, None]).astype(jnp.int32),
                 axis=1)
    active = (tile_starts < pcum[E - 1]).astype(jnp.int32)
    te = jnp.minimum(te, E - 1).astype(jnp.int32)
    return dest, wwide, invr, te, active


def _sc_dispatch(xb32, dest3, wwide):
    D2 = D // 2
    spw = (K * N) // NW
    nch = spw // SCH
    mesh = plsc.VectorSubcoreMesh(core_axis_name="c", subcore_axis_name="s")

    @functools.partial(
        pl.kernel,
        out_type=[
            jax.ShapeDtypeStruct((NPAD, D2), jnp.int32),
            jax.ShapeDtypeStruct((NPAD, 128), jnp.float32),
        ],
        mesh=mesh,
        scratch_types=[
            pltpu.VMEM((SCH,), jnp.int32),
            pltpu.VMEM((SCH, D2), jnp.int32),
            pltpu.VMEM((SCH, 128), jnp.float32),
            pltpu.SemaphoreType.DMA,
            pltpu.SemaphoreType.DMA,
        ],
    )
    def k(x_hbm, d_hbm, w_hbm, xs_hbm, ws_hbm, idx_v, rows_v, w_v, s0, s1):
        wid = lax.axis_index("s") * 2 + lax.axis_index("c")
        base = wid * spw
        tok = base - (base // N) * N
        for c in range(nch):
            off = c * SCH
            pltpu.sync_copy(d_hbm.at[wid, c], idx_v)
            pltpu.sync_copy(x_hbm.at[pl.ds(tok + off, SCH)], rows_v)
            pltpu.sync_copy(w_hbm.at[pl.ds(base + off, SCH)], w_v)
            cpx = pltpu.async_copy(rows_v, xs_hbm.at[idx_v], s0)
            cpw = pltpu.async_copy(w_v, ws_hbm.at[idx_v], s1)
            cpx.wait()
            cpw.wait()

    return k(xb32, dest3, wwide)


def _expert_body(te_ref, act_ref, xs_ref, w1_ref, w2_ref, ws_ref, ys_ref):
    t = pl.program_id(0)

    @pl.when(act_ref[t] == 1)
    def _():
        xt = xs_ref[...]
        g = lax.dot_general(
            xt, w1_ref[0, 0], (((1,), (1,)), ((), ())),
            preferred_element_type=jnp.float32)
        l = lax.dot_general(
            xt, w1_ref[0, 1], (((1,), (1,)), ((), ())),
            preferred_element_type=jnp.float32)
        a = (g * lax.logistic(g) * l).astype(jnp.bfloat16)
        oe = lax.dot_general(
            a, w2_ref[0], (((1,), (1,)), ((), ())),
            preferred_element_type=jnp.float32)
        ys_ref[...] = ws_ref[...] * oe

    @pl.when(act_ref[t] == 0)
    def _():
        ys_ref[...] = jnp.zeros_like(ys_ref)


def _experts(xs, W1r, W2, wslot, te, active):
    grid_spec = pltpu.PrefetchScalarGridSpec(
        num_scalar_prefetch=2,
        grid=(NT2,),
        in_specs=[
            pl.BlockSpec((TT, D), lambda t, te_r, ac_r: (t, 0)),
            pl.BlockSpec((1, 2, H, D), lambda t, te_r, ac_r: (te_r[t], 0, 0, 0)),
            pl.BlockSpec((1, D, H), lambda t, te_r, ac_r: (te_r[t], 0, 0)),
            pl.BlockSpec((TT, 1), lambda t, te_r, ac_r: (t, 0)),
        ],
        out_specs=pl.BlockSpec((TT, D), lambda t, te_r, ac_r: (t, 0)),
    )
    return pl.pallas_call(
        _expert_body,
        grid_spec=grid_spec,
        out_shape=jax.ShapeDtypeStruct((NPAD, D), jnp.float32),
    )(te, active, xs, W1r, W2, wslot)


def _sc_combine(ys, i0, i1):
    tpw = N // NW
    nch = tpw // CCH
    mesh = plsc.VectorSubcoreMesh(core_axis_name="c", subcore_axis_name="s")

    @functools.partial(
        pl.kernel,
        out_type=jax.ShapeDtypeStruct((N, D), jnp.float32),
        mesh=mesh,
        scratch_types=[
            [pltpu.VMEM((CCH,), jnp.int32) for _ in range(2)],
            [pltpu.VMEM((CCH,), jnp.int32) for _ in range(2)],
            [pltpu.VMEM((CCH, D), jnp.float32) for _ in range(2)],
            [pltpu.VMEM((CCH, D), jnp.float32) for _ in range(2)],
            [pltpu.SemaphoreType.DMA for _ in range(2)],
            [pltpu.SemaphoreType.DMA for _ in range(2)],
            [pltpu.SemaphoreType.DMA for _ in range(2)],
        ],
    )
    def k(ys_hbm, i0_hbm, i1_hbm, out_hbm, i0_v, i1_v, g0, g1, s0, s1, sw):
        wid = lax.axis_index("s") * 2 + lax.axis_index("c")
        base = wid * tpw
        cp0 = [None] * nch
        cp1 = [None] * nch
        wb = [None] * nch
        for c in range(nch):
            b = c & 1
            off = base + c * CCH
            pltpu.sync_copy(i0_hbm.at[pl.ds(off, CCH)], i0_v[b])
            pltpu.sync_copy(i1_hbm.at[pl.ds(off, CCH)], i1_v[b])
            cp0[c] = pltpu.async_copy(ys_hbm.at[i0_v[b]], g0[b], s0[b])
            cp1[c] = pltpu.async_copy(ys_hbm.at[i1_v[b]], g1[b], s1[b])
        for c in range(nch):
            b = c & 1
            off = base + c * CCH
            cp0[c].wait()
            cp1[c].wait()

            def add_row(r, carry):
                for j in range(D // 16):
                    sl = pl.ds(j * 16, 16)
                    g0[b][r, sl] = g0[b][r, sl] + g1[b][r, sl]
                return carry

            lax.fori_loop(0, CCH, add_row, 0)
            wb[c] = pltpu.async_copy(g0[b], out_hbm.at[pl.ds(off, CCH)], sw[b])
        for c in range(nch):
            wb[c].wait()

    return k(ys, i0, i1)


@jax.jit
def kernel(x, Wg, W1, W2):
    xf = x.reshape(N, D)
    logitsT, selT, rwT = _router(xf, Wg)
    dest, wwide, invr, te, active = _dispatch_meta(selT, rwT)
    xb = xf.astype(jnp.bfloat16)
    xb32 = lax.bitcast_convert_type(xb.reshape(N, D // 2, 2), jnp.int32)
    dest3 = dest.reshape(NW, (K * N) // (NW * SCH), SCH)
    xs32, ws = _sc_dispatch(xb32, dest3, wwide)
    xs = lax.bitcast_convert_type(xs32, jnp.bfloat16).reshape(NPAD, D)
    wslot = lax.slice(ws, (0, 0), (NPAD, 1))
    W1b = W1.reshape(E, 2, H, D).astype(jnp.bfloat16)
    W2b = W2.astype(jnp.bfloat16)
    ys = _experts(xs, W1b, W2b, wslot, te, active)
    out = _sc_combine(ys, invr[0], invr[1])
    return out.reshape(B, S, D), logitsT.T.reshape(B, S, E)


# speedup vs baseline: 3.3124x; 2.4730x over previous
import functools

import jax
import jax.numpy as jnp
from jax import lax
from jax.experimental import pallas as pl
from jax.experimental.pallas import tpu as pltpu
from jax.experimental.pallas import tpu_sc as plsc

B, S, D = 1, 2048, 1024
E, K, H = 8, 2, 1024
N = B * S
TT = 256
NPAD = N * K + E * TT
NT2 = NPAD // TT
NEG = -1e30

NW = 32
SCH = 64
CCH = 16


def _router_body(wg_ref, x_ref, logits_ref, sel_ref, rw_ref):
    xt = x_ref[...]
    lt = lax.dot_general(
        wg_ref[...], xt, (((1,), (1,)), ((), ())),
        preferred_element_type=jnp.float32)
    idx = lax.broadcasted_iota(jnp.int32, (E, TT), 0)
    m1 = jnp.max(lt, axis=0, keepdims=True)
    a1 = jnp.min(jnp.where(lt == m1, idx, E), axis=0, keepdims=True)
    lt2 = jnp.where(idx == a1, NEG, lt)
    m2 = jnp.max(lt2, axis=0, keepdims=True)
    a2 = jnp.min(jnp.where(lt2 == m2, idx, E), axis=0, keepdims=True)
    e2 = jnp.exp(m2 - m1)
    denom = 1.0 + e2
    logits_ref[...] = lt
    sel_ref[...] = jnp.concatenate([a1, a2], axis=0)
    rw_ref[...] = jnp.concatenate([1.0 / denom, e2 / denom], axis=0)


def _router(xf, Wg):
    nt = N // TT
    return pl.pallas_call(
        _router_body,
        grid=(nt,),
        in_specs=[
            pl.BlockSpec((E, D), lambda t: (0, 0)),
            pl.BlockSpec((TT, D), lambda t: (t, 0)),
        ],
        out_specs=[
            pl.BlockSpec((E, TT), lambda t: (0, t)),
            pl.BlockSpec((K, TT), lambda t: (0, t)),
            pl.BlockSpec((K, TT), lambda t: (0, t)),
        ],
        out_shape=[
            jax.ShapeDtypeStruct((E, N), jnp.float32),
            jax.ShapeDtypeStruct((K, N), jnp.int32),
            jax.ShapeDtypeStruct((K, N), jnp.float32),
        ],
    )(Wg, xf)


def _dispatch_meta(selT, rwT):
    sel_flat = selT.reshape(-1)
    w_flat = rwT.reshape(-1)
    onehot = (sel_flat[:, None] == jnp.arange(E, dtype=jnp.int32)[None, :])
    oh = onehot.astype(jnp.int32)
    counts = jnp.sum(oh, axis=0)
    rank = jnp.sum((jnp.cumsum(oh, axis=0) - 1) * oh, axis=1)
    pcounts = ((counts + TT - 1) // TT) * TT
    pcum = jnp.cumsum(pcounts)
    pstart = pcum - pcounts
    pstart_sel = jnp.sum(oh * pstart[None, :], axis=1)
    dest = (pstart_sel + rank).astype(jnp.int32)
    invr = dest.reshape(K, N)
    wwide = jnp.broadcast_to(w_flat[:, None], (K * N, 128))
    tile_starts = jnp.arange(NT2, dtype=jnp.int32) * TT
    te = jnp.sum((pcum[None, :] <= tile_starts[:, None]).astype(jnp.int32),
                 axis=1)
    active = (tile_starts < pcum[E - 1]).astype(jnp.int32)
    te = jnp.minimum(te, E - 1).astype(jnp.int32)
    return dest, wwide, invr, te, active


def _sc_dispatch(xf, dest3, wwide):
    D2 = D
    spw = (K * N) // NW
    nch = spw // SCH
    mesh = plsc.VectorSubcoreMesh(core_axis_name="c", subcore_axis_name="s")

    @functools.partial(
        pl.kernel,
        out_type=[
            jax.ShapeDtypeStruct((NPAD, D2), jnp.float32),
            jax.ShapeDtypeStruct((NPAD, 128), jnp.float32),
        ],
        mesh=mesh,
        scratch_types=[
            pltpu.VMEM((SCH,), jnp.int32),
            pltpu.VMEM((SCH, D2), jnp.float32),
            pltpu.VMEM((SCH, 128), jnp.float32),
            pltpu.SemaphoreType.DMA,
            pltpu.SemaphoreType.DMA,
        ],
    )
    def k(x_hbm, d_hbm, w_hbm, xs_hbm, ws_hbm, idx_v, rows_v, w_v, s0, s1):
        wid = lax.axis_index("s") * 2 + lax.axis_index("c")
        base = wid * spw
        tok = base - (base // N) * N
        for c in range(nch):
            off = c * SCH
            pltpu.sync_copy(d_hbm.at[wid, c], idx_v)
            pltpu.sync_copy(x_hbm.at[pl.ds(tok + off, SCH)], rows_v)
            pltpu.sync_copy(w_hbm.at[pl.ds(base + off, SCH)], w_v)
            cpx = pltpu.async_copy(rows_v, xs_hbm.at[idx_v], s0)
            cpw = pltpu.async_copy(w_v, ws_hbm.at[idx_v], s1)
            cpx.wait()
            cpw.wait()

    return k(xf, dest3, wwide)


def _expert_body(te_ref, act_ref, xs_ref, w1_ref, w2_ref, ws_ref, ys_ref):
    t = pl.program_id(0)

    @pl.when(act_ref[t] == 1)
    def _():
        xt = xs_ref[...]
        g = lax.dot_general(
            xt, w1_ref[0, 0], (((1,), (1,)), ((), ())),
            preferred_element_type=jnp.float32)
        l = lax.dot_general(
            xt, w1_ref[0, 1], (((1,), (1,)), ((), ())),
            preferred_element_type=jnp.float32)
        a = g * lax.logistic(g) * l
        oe = lax.dot_general(
            a, w2_ref[0], (((1,), (1,)), ((), ())),
            preferred_element_type=jnp.float32)
        ys_ref[...] = ws_ref[...] * oe

    @pl.when(act_ref[t] == 0)
    def _():
        ys_ref[...] = jnp.zeros_like(ys_ref)


def _experts(xs, W1r, W2, wslot, te, active):
    grid_spec = pltpu.PrefetchScalarGridSpec(
        num_scalar_prefetch=2,
        grid=(NT2,),
        in_specs=[
            pl.BlockSpec((TT, D), lambda t, te_r, ac_r: (t, 0)),
            pl.BlockSpec((1, 2, H, D), lambda t, te_r, ac_r: (te_r[t], 0, 0, 0)),
            pl.BlockSpec((1, D, H), lambda t, te_r, ac_r: (te_r[t], 0, 0)),
            pl.BlockSpec((TT, 1), lambda t, te_r, ac_r: (t, 0)),
        ],
        out_specs=pl.BlockSpec((TT, D), lambda t, te_r, ac_r: (t, 0)),
    )
    return pl.pallas_call(
        _expert_body,
        grid_spec=grid_spec,
        out_shape=jax.ShapeDtypeStruct((NPAD, D), jnp.float32),
    )(te, active, xs, W1r, W2, wslot)


def _sc_combine(ys, i0, i1):
    tpw = N // NW
    nch = tpw // CCH
    mesh = plsc.VectorSubcoreMesh(core_axis_name="c", subcore_axis_name="s")

    @functools.partial(
        pl.kernel,
        out_type=jax.ShapeDtypeStruct((N, D), jnp.float32),
        mesh=mesh,
        scratch_types=[
            [pltpu.VMEM((CCH,), jnp.int32) for _ in range(2)],
            [pltpu.VMEM((CCH,), jnp.int32) for _ in range(2)],
            [pltpu.VMEM((CCH, D), jnp.float32) for _ in range(2)],
            [pltpu.VMEM((CCH, D), jnp.float32) for _ in range(2)],
            [pltpu.VMEM((CCH, D), jnp.float32) for _ in range(2)],
            [pltpu.SemaphoreType.DMA for _ in range(2)],
            [pltpu.SemaphoreType.DMA for _ in range(2)],
            [pltpu.SemaphoreType.DMA for _ in range(2)],
        ],
    )
    def k(ys_hbm, i0_hbm, i1_hbm, out_hbm, i0_v, i1_v, g0, g1, o, s0, s1, sw):
        wid = lax.axis_index("s") * 2 + lax.axis_index("c")
        base = wid * tpw
        cp0 = [None] * nch
        cp1 = [None] * nch
        wb = [None] * nch

        def start(c):
            b = c & 1
            off = base + c * CCH
            pltpu.sync_copy(i0_hbm.at[pl.ds(off, CCH)], i0_v[b])
            pltpu.sync_copy(i1_hbm.at[pl.ds(off, CCH)], i1_v[b])
            cp0[c] = pltpu.async_copy(ys_hbm.at[i0_v[b]], g0[b], s0[b])
            cp1[c] = pltpu.async_copy(ys_hbm.at[i1_v[b]], g1[b], s1[b])

        start(0)
        if nch > 1:
            start(1)
        for c in range(nch):
            b = c & 1
            cp0[c].wait()
            cp1[c].wait()
            if c >= 2:
                wb[c - 2].wait()

            def add_row(r, carry):
                for j in range(D // 16):
                    sl = pl.ds(j * 16, 16)
                    o[b][r, sl] = g0[b][r, sl] + g1[b][r, sl]
                return carry

            lax.fori_loop(0, CCH, add_row, 0)
            wb[c] = pltpu.async_copy(
                o[b], out_hbm.at[pl.ds(base + c * CCH, CCH)], sw[b])
            if c + 2 < nch:
                start(c + 2)
        wb[nch - 1].wait()
        if nch > 1:
            wb[nch - 2].wait()

    return k(ys, i0, i1)


@jax.jit
def kernel(x, Wg, W1, W2):
    xf = x.reshape(N, D)
    logitsT, selT, rwT = _router(xf, Wg)
    dest, wwide, invr, te, active = _dispatch_meta(selT, rwT)
    dest3 = dest.reshape(NW, (K * N) // (NW * SCH), SCH)
    xs, ws = _sc_dispatch(xf, dest3, wwide)
    wslot = lax.slice(ws, (0, 0), (NPAD, 1))
    W1r = W1.reshape(E, 2, H, D)
    ys = _experts(xs, W1r, W2, wslot, te, active)
    out = _sc_combine(ys, invr[0], invr[1])
    return out.reshape(B, S, D), logitsT.T.reshape(B, S, E)


# speedup vs baseline: 3.3414x; 1.0087x over previous
import functools

import jax
import jax.numpy as jnp
from jax import lax
from jax.experimental import pallas as pl
from jax.experimental.pallas import tpu as pltpu
from jax.experimental.pallas import tpu_sc as plsc

B, S, D = 1, 2048, 1024
E, K, H = 8, 2, 1024
N = B * S
TT = 256
NPAD = N * K + E * TT
NT2 = NPAD // TT
NEG = -1e30

NW = 32
SCH = 64
CCH = 16


def _router_body(wg_ref, x_ref, logits_ref, sel_ref, rw_ref):
    xt = x_ref[...]
    lt = lax.dot_general(
        wg_ref[...], xt, (((1,), (1,)), ((), ())),
        preferred_element_type=jnp.float32)
    idx = lax.broadcasted_iota(jnp.int32, (E, TT), 0)
    m1 = jnp.max(lt, axis=0, keepdims=True)
    a1 = jnp.min(jnp.where(lt == m1, idx, E), axis=0, keepdims=True)
    lt2 = jnp.where(idx == a1, NEG, lt)
    m2 = jnp.max(lt2, axis=0, keepdims=True)
    a2 = jnp.min(jnp.where(lt2 == m2, idx, E), axis=0, keepdims=True)
    e2 = jnp.exp(m2 - m1)
    denom = 1.0 + e2
    logits_ref[...] = lt
    sel_ref[...] = jnp.concatenate([a1, a2], axis=0)
    rw = jnp.concatenate([1.0 / denom, e2 / denom], axis=0)
    rw_ref[...] = jnp.broadcast_to(rw[:, :, None], (K, rw.shape[1], 128))


def _router(xf, Wg):
    nt = N // TT
    return pl.pallas_call(
        _router_body,
        grid=(nt,),
        in_specs=[
            pl.BlockSpec((E, D), lambda t: (0, 0)),
            pl.BlockSpec((TT, D), lambda t: (t, 0)),
        ],
        out_specs=[
            pl.BlockSpec((E, TT), lambda t: (0, t)),
            pl.BlockSpec((K, TT), lambda t: (0, t)),
            pl.BlockSpec((K, TT, 128), lambda t: (0, t, 0)),
        ],
        out_shape=[
            jax.ShapeDtypeStruct((E, N), jnp.float32),
            jax.ShapeDtypeStruct((K, N), jnp.int32),
            jax.ShapeDtypeStruct((K, N, 128), jnp.float32),
        ],
    )(Wg, xf)


def _dispatch_meta(selT):
    sel_flat = selT.reshape(-1)
    onehot = (sel_flat[:, None] == jnp.arange(E, dtype=jnp.int32)[None, :])
    oh = onehot.astype(jnp.int32)
    counts = jnp.sum(oh, axis=0)
    rank = jnp.sum((jnp.cumsum(oh, axis=0) - 1) * oh, axis=1)
    pcounts = ((counts + TT - 1) // TT) * TT
    pcum = jnp.cumsum(pcounts)
    pstart = pcum - pcounts
    pstart_sel = jnp.sum(oh * pstart[None, :], axis=1)
    dest = (pstart_sel + rank).astype(jnp.int32)
    invr = dest.reshape(K, N)
    tile_starts = jnp.arange(NT2, dtype=jnp.int32) * TT
    te = jnp.sum((pcum[None, :] <= tile_starts[:, None]).astype(jnp.int32),
                 axis=1)
    active = (tile_starts < pcum[E - 1]).astype(jnp.int32)
    te = jnp.minimum(te, E - 1).astype(jnp.int32)
    return dest, invr, te, active


def _sc_dispatch(xf, dest3, wwide):
    D2 = D
    spw = (K * N) // NW
    nch = spw // SCH
    mesh = plsc.VectorSubcoreMesh(core_axis_name="c", subcore_axis_name="s")

    @functools.partial(
        pl.kernel,
        out_type=[
            jax.ShapeDtypeStruct((NPAD, D2), jnp.float32),
            jax.ShapeDtypeStruct((NPAD, 128), jnp.float32),
        ],
        mesh=mesh,
        scratch_types=[
            pltpu.VMEM((SCH,), jnp.int32),
            pltpu.VMEM((SCH, D2), jnp.float32),
            pltpu.VMEM((SCH, 128), jnp.float32),
            pltpu.SemaphoreType.DMA,
            pltpu.SemaphoreType.DMA,
        ],
    )
    def k(x_hbm, d_hbm, w_hbm, xs_hbm, ws_hbm, idx_v, rows_v, w_v, s0, s1):
        wid = lax.axis_index("s") * 2 + lax.axis_index("c")
        base = wid * spw
        tok = base - (base // N) * N
        for c in range(nch):
            off = c * SCH
            pltpu.sync_copy(d_hbm.at[wid, c], idx_v)
            pltpu.sync_copy(x_hbm.at[pl.ds(tok + off, SCH)], rows_v)
            pltpu.sync_copy(w_hbm.at[pl.ds(base + off, SCH)], w_v)
            cpx = pltpu.async_copy(rows_v, xs_hbm.at[idx_v], s0)
            cpw = pltpu.async_copy(w_v, ws_hbm.at[idx_v], s1)
            cpx.wait()
            cpw.wait()

    return k(xf, dest3, wwide)


def _expert_body(te_ref, act_ref, xs_ref, w1_ref, w2_ref, ws_ref, ys_ref):
    t = pl.program_id(0)

    @pl.when(act_ref[t] == 1)
    def _():
        xt = xs_ref[...]
        g = lax.dot_general(
            xt, w1_ref[0, 0], (((1,), (1,)), ((), ())),
            preferred_element_type=jnp.float32)
        l = lax.dot_general(
            xt, w1_ref[0, 1], (((1,), (1,)), ((), ())),
            preferred_element_type=jnp.float32)
        a = g * lax.logistic(g) * l
        oe = lax.dot_general(
            a, w2_ref[0], (((1,), (1,)), ((), ())),
            preferred_element_type=jnp.float32)
        ys_ref[...] = ws_ref[:, 0:1] * oe

    @pl.when(act_ref[t] == 0)
    def _():
        ys_ref[...] = jnp.zeros_like(ys_ref)


def _experts(xs, W1r, W2, wslot, te, active):
    grid_spec = pltpu.PrefetchScalarGridSpec(
        num_scalar_prefetch=2,
        grid=(NT2,),
        in_specs=[
            pl.BlockSpec((TT, D), lambda t, te_r, ac_r: (t, 0)),
            pl.BlockSpec((1, 2, H, D), lambda t, te_r, ac_r: (te_r[t], 0, 0, 0)),
            pl.BlockSpec((1, D, H), lambda t, te_r, ac_r: (te_r[t], 0, 0)),
            pl.BlockSpec((TT, 128), lambda t, te_r, ac_r: (t, 0)),
        ],
        out_specs=pl.BlockSpec((TT, D), lambda t, te_r, ac_r: (t, 0)),
    )
    return pl.pallas_call(
        _expert_body,
        grid_spec=grid_spec,
        out_shape=jax.ShapeDtypeStruct((NPAD, D), jnp.float32),
    )(te, active, xs, W1r, W2, wslot)


def _sc_combine(ys, i0, i1):
    tpw = N // NW
    nch = tpw // CCH
    mesh = plsc.VectorSubcoreMesh(core_axis_name="c", subcore_axis_name="s")

    @functools.partial(
        pl.kernel,
        out_type=jax.ShapeDtypeStruct((N, D), jnp.float32),
        mesh=mesh,
        scratch_types=[
            [pltpu.VMEM((CCH,), jnp.int32) for _ in range(2)],
            [pltpu.VMEM((CCH,), jnp.int32) for _ in range(2)],
            [pltpu.VMEM((CCH, D), jnp.float32) for _ in range(2)],
            [pltpu.VMEM((CCH, D), jnp.float32) for _ in range(2)],
            [pltpu.VMEM((CCH, D), jnp.float32) for _ in range(2)],
            [pltpu.SemaphoreType.DMA for _ in range(2)],
            [pltpu.SemaphoreType.DMA for _ in range(2)],
            [pltpu.SemaphoreType.DMA for _ in range(2)],
        ],
    )
    def k(ys_hbm, i0_hbm, i1_hbm, out_hbm, i0_v, i1_v, g0, g1, o, s0, s1, sw):
        wid = lax.axis_index("s") * 2 + lax.axis_index("c")
        base = wid * tpw
        cp0 = [None] * nch
        cp1 = [None] * nch
        wb = [None] * nch

        def start(c):
            b = c & 1
            off = base + c * CCH
            pltpu.sync_copy(i0_hbm.at[pl.ds(off, CCH)], i0_v[b])
            pltpu.sync_copy(i1_hbm.at[pl.ds(off, CCH)], i1_v[b])
            cp0[c] = pltpu.async_copy(ys_hbm.at[i0_v[b]], g0[b], s0[b])
            cp1[c] = pltpu.async_copy(ys_hbm.at[i1_v[b]], g1[b], s1[b])

        start(0)
        if nch > 1:
            start(1)
        for c in range(nch):
            b = c & 1
            cp0[c].wait()
            cp1[c].wait()
            if c >= 2:
                wb[c - 2].wait()

            def add_row(r, carry):
                for j in range(D // 16):
                    sl = pl.ds(j * 16, 16)
                    o[b][r, sl] = g0[b][r, sl] + g1[b][r, sl]
                return carry

            lax.fori_loop(0, CCH, add_row, 0)
            wb[c] = pltpu.async_copy(
                o[b], out_hbm.at[pl.ds(base + c * CCH, CCH)], sw[b])
            if c + 2 < nch:
                start(c + 2)
        wb[nch - 1].wait()
        if nch > 1:
            wb[nch - 2].wait()

    return k(ys, i0, i1)


@jax.jit
def kernel(x, Wg, W1, W2):
    xf = x.reshape(N, D)
    logitsT, selT, rww = _router(xf, Wg)
    dest, invr, te, active = _dispatch_meta(selT)
    dest3 = dest.reshape(NW, (K * N) // (NW * SCH), SCH)
    xs, ws = _sc_dispatch(xf, dest3, rww.reshape(K * N, 128))
    W1r = W1.reshape(E, 2, H, D)
    ys = _experts(xs, W1r, W2, ws, te, active)
    out = _sc_combine(ys, invr[0], invr[1])
    return out.reshape(B, S, D), logitsT.T.reshape(B, S, E)
